# baseline (speedup 1.0000x reference)
import functools

import jax
import jax.numpy as jnp
from jax import lax
from jax.experimental import pallas as pl
from jax.experimental.pallas import tpu as pltpu
from jax.experimental.pallas import tpu_sc as plsc

N = 10000
E = 160000
F = 256
H = 256
HH = 128
P = 64
C = 2
G = 16

NC = 2
NS = 16
NP = 10240
EP = 163840
CH = 80
ROWS_PER_SUB = NP // NS
EDGES_PER_SUB = EP // NS
EDGES_PER_TILE = EP // (NC * NS)
NCH = EDGES_PER_SUB // CH
DW = 16

@functools.cache
def _sc_kernels():
    mesh = plsc.VectorSubcoreMesh(core_axis_name="c", subcore_axis_name="s",
                                  num_cores=NC, num_subcores=NS)

    @functools.partial(
        pl.kernel,
        out_type=jax.ShapeDtypeStruct((NC, NP, DW), jnp.float32),
        mesh=mesh,
        scratch_types=[
            pltpu.VMEM((EDGES_PER_TILE // CH, CH), jnp.int32),
            pltpu.VMEM((CH, DW), jnp.float32),
            pltpu.VMEM_SHARED((NP, DW), jnp.float32),
            pltpu.SemaphoreType.DMA,
            pltpu.SemaphoreType.DMA,
        ],
    )
    def _deg_kernel(dstd_hbm, zeros_hbm, ones_hbm, deg_out, didx_v, ones_v,
                    deg_sh, dsem0, dsem1):
        c = lax.axis_index("c")
        s = lax.axis_index("s")
        wid = s * NC + c
        pltpu.sync_copy(zeros_hbm.at[pl.ds(s * ROWS_PER_SUB, ROWS_PER_SUB)],
                        deg_sh.at[pl.ds(s * ROWS_PER_SUB, ROWS_PER_SUB)])
        pltpu.sync_copy(ones_hbm, ones_v)
        plsc.subcore_barrier()

        nd = EDGES_PER_TILE // CH
        pltpu.sync_copy(dstd_hbm.at[wid], didx_v)
        sd = [None] * nd
        for j in range(nd):
            if j >= 2:
                sd[j - 2].wait()
            sd[j] = pltpu.async_copy(ones_v, deg_sh.at[didx_v.at[j]],
                                     dsem0 if j % 2 == 0 else dsem1, add=True)
        sd[nd - 2].wait()
        sd[nd - 1].wait()
        plsc.subcore_barrier()
        pltpu.sync_copy(deg_sh.at[pl.ds(s * ROWS_PER_SUB, ROWS_PER_SUB)],
                        deg_out.at[c, pl.ds(s * ROWS_PER_SUB, ROWS_PER_SUB)])

    @functools.partial(
        pl.kernel,
        out_type=jax.ShapeDtypeStruct((NC, NP, HH), jnp.float32),
        mesh=mesh,
        scratch_types=[
            pltpu.VMEM((NCH // 4, CH), jnp.int32),
            pltpu.VMEM((NCH // 4, CH), jnp.int32),
            pltpu.VMEM((CH, HH), jnp.float32),
            pltpu.VMEM((CH, HH), jnp.float32),
            pltpu.VMEM((CH, HH), jnp.float32),
            pltpu.VMEM((CH, HH), jnp.float32),
            pltpu.VMEM_SHARED((NP, HH), jnp.float32),
        ] + [pltpu.SemaphoreType.DMA] * 8,
    )
    def _edge_kernel(gtab_hbm, src4_hbm, dst3_hbm, zrows_hbm, agg_out,
                     sidx, didx, buf0, buf1, buf2, buf3, agg_sh,
                     gsem0, gsem1, gsem2, gsem3, ssem0, ssem1, ssem2, ssem3):
        c = lax.axis_index("c")
        s = lax.axis_index("s")
        nh = NCH // 4
        bufs = (buf0, buf1, buf2, buf3)
        gsems = (gsem0, gsem1, gsem2, gsem3)
        ssems = (ssem0, ssem1, ssem2, ssem3)
        pltpu.sync_copy(zrows_hbm.at[pl.ds(s * ROWS_PER_SUB, ROWS_PER_SUB)],
                        agg_sh.at[pl.ds(s * ROWS_PER_SUB, ROWS_PER_SUB)])
        plsc.subcore_barrier()

        for part in range(4):
            pltpu.sync_copy(src4_hbm.at[c, s, pl.ds(part * nh, nh)], sidx)
            pltpu.sync_copy(dst3_hbm.at[s, pl.ds(part * nh, nh)], didx)
            gd = [None] * nh
            sd = [None] * nh
            for j in range(nh):
                b = j % 4
                if j >= 4:
                    sd[j - 4].wait()
                gd[j] = pltpu.async_copy(gtab_hbm.at[sidx.at[j]],
                                         bufs[b], gsems[b])
                if j >= 1:
                    gd[j - 1].wait()
                    sd[j - 1] = pltpu.async_copy(
                        bufs[(j - 1) % 4], agg_sh.at[didx.at[j - 1]],
                        ssems[(j - 1) % 4], add=True)
            gd[nh - 1].wait()
            sd[nh - 1] = pltpu.async_copy(
                bufs[(nh - 1) % 4], agg_sh.at[didx.at[nh - 1]],
                ssems[(nh - 1) % 4], add=True)
            for k in range(4):
                sd[nh - 4 + k].wait()
        plsc.subcore_barrier()
        pltpu.sync_copy(agg_sh.at[pl.ds(s * ROWS_PER_SUB, ROWS_PER_SUB)],
                        agg_out.at[c, pl.ds(s * ROWS_PER_SUB, ROWS_PER_SUB)])

    return _deg_kernel, _edge_kernel


BN = 512
NB = NP // BN


def _di_block(deg0_ref, deg1_ref):
    deg = deg0_ref[...] + deg1_ref[...]
    return lax.rsqrt(jnp.maximum(deg, 1.0))


def _front_body(x_ref, deg0_ref, deg1_ref, wa_ref, wb_ref, b_ref, m_ref, z_ref):
    di = _di_block(deg0_ref, deg1_ref)
    xb = x_ref[...]
    g = xb * di
    m_ref[0] = jnp.dot(g, wa_ref[:, :HH], preferred_element_type=jnp.float32)
    m_ref[1] = jnp.dot(g, wa_ref[:, HH:], preferred_element_type=jnp.float32)
    z_ref[...] = jnp.dot(xb, wb_ref[...], preferred_element_type=jnp.float32) + b_ref[...]


def _mid_body(agg_ref, z_ref, deg0_ref, deg1_ref, wa_ref, wb_ref, b_ref,
              m_ref, zn_ref):
    di = _di_block(deg0_ref, deg1_ref)
    a = jnp.concatenate([agg_ref[0], agg_ref[1]], axis=-1)
    h = jax.nn.relu(a * di + z_ref[...])
    g = h * di
    m_ref[0] = jnp.dot(g, wa_ref[:, :HH], preferred_element_type=jnp.float32)
    m_ref[1] = jnp.dot(g, wa_ref[:, HH:], preferred_element_type=jnp.float32)
    zn_ref[...] = jnp.dot(h, wb_ref[...], preferred_element_type=jnp.float32) + b_ref[...]


def _final_body(agg_ref, z_ref, deg0_ref, deg1_ref, gid_ref,
                wfc_ref, bfc_ref, wout_ref, bout_ref, out_ref, acc):
    n = pl.program_id(0)
    di = _di_block(deg0_ref, deg1_ref)
    a = jnp.concatenate([agg_ref[0], agg_ref[1]], axis=-1)
    h = jax.nn.relu(a * di + z_ref[...])
    gid = gid_ref[...]
    iota = lax.broadcasted_iota(jnp.int32, (G, BN), 0)
    oneh = (iota == gid).astype(jnp.float32)
    part = jnp.dot(oneh, h, preferred_element_type=jnp.float32)

    @pl.when(n == 0)
    def _():
        acc[...] = part

    @pl.when(n > 0)
    def _():
        acc[...] = acc[...] + part

    @pl.when(n == NB - 1)
    def _():
        pooled = acc[...]
        t = jnp.dot(pooled, wfc_ref[...], preferred_element_type=jnp.float32) + bfc_ref[...]
        t = jnp.dot(t, wout_ref[...], preferred_element_type=jnp.float32) + bout_ref[...]
        out_ref[...] = 1.0 / (1.0 + jnp.exp(-t))


def _node_spec(width):
    return pl.BlockSpec((BN, width), lambda n: (n, 0))


def _half_spec():
    return pl.BlockSpec((NC, BN, HH), lambda n: (0, n, 0))


def _full_spec(shape):
    nd = len(shape)
    return pl.BlockSpec(shape, lambda n: (0,) * nd)


def _tc_front(x_p, deg0, deg1, Wa, Wb, b):
    return pl.pallas_call(
        _front_body,
        grid=(NB,),
        in_specs=[
            _node_spec(F), _node_spec(1), _node_spec(1),
            _full_spec((F, H)), _full_spec((F, H)), _full_spec((1, H)),
        ],
        out_specs=[_half_spec(), _node_spec(H)],
        out_shape=[
            jax.ShapeDtypeStruct((NC, NP, HH), jnp.float32),
            jax.ShapeDtypeStruct((NP, H), jnp.float32),
        ],
    )(x_p, deg0, deg1, Wa, Wb, b.reshape(1, H))


def _tc_mid(agg, z, deg0, deg1, Wa, Wb, b):
    return pl.pallas_call(
        _mid_body,
        grid=(NB,),
        in_specs=[
            _half_spec(), _node_spec(H), _node_spec(1), _node_spec(1),
            _full_spec((H, H)), _full_spec((H, H)), _full_spec((1, H)),
        ],
        out_specs=[_half_spec(), _node_spec(H)],
        out_shape=[
            jax.ShapeDtypeStruct((NC, NP, HH), jnp.float32),
            jax.ShapeDtypeStruct((NP, H), jnp.float32),
        ],
    )(agg, z, deg0, deg1, Wa, Wb, b.reshape(1, H))


def _tc_final(agg, z, deg0, deg1, gid, Wfc, bfc, Wout, bout):
    return pl.pallas_call(
        _final_body,
        grid=(NB,),
        in_specs=[
            _half_spec(), _node_spec(H), _node_spec(1), _node_spec(1),
            pl.BlockSpec((1, BN), lambda n: (0, n)),
            _full_spec((H, P)), _full_spec((1, P)),
            _full_spec((P, C)), _full_spec((1, C)),
        ],
        out_specs=_full_spec((G, C)),
        out_shape=jax.ShapeDtypeStruct((G, C), jnp.float32),
        scratch_shapes=[pltpu.VMEM((G, H), jnp.float32)],
        compiler_params=pltpu.CompilerParams(
            dimension_semantics=("arbitrary",)),
    )(agg, z, deg0, deg1, gid, Wfc, bfc.reshape(1, P), Wout, bout.reshape(1, C))


def kernel(x, edge_index, i, W1a, W1b, b1, W2a, W2b, b2, W3a, W3b, b3,
           Wfc, bfc, Wout, bout):
    f32 = jnp.float32
    src = edge_index[0]
    dst = edge_index[1]
    pad_e = EP - E
    src_p = jnp.concatenate([src, jnp.zeros((pad_e,), jnp.int32)])
    dst_p = jnp.concatenate([dst, jnp.full((pad_e,), N, jnp.int32)])
    src4 = jnp.stack([src_p, src_p + NP]).reshape(NC, NS, NCH, CH)
    dst3 = dst_p.reshape(NS, NCH, CH)
    x_p = jnp.pad(x, ((0, NP - N), (0, 0)))
    gid = jnp.pad(i, (0, NP - N), constant_values=G).reshape(1, NP)

    zrows = jnp.zeros((NP, HH), f32)
    zdeg = jnp.zeros((NP, DW), f32)
    ones_c = jnp.ones((CH, DW), f32)

    _deg_kernel, _edge_kernel = _sc_kernels()
    dstd = dst_p.reshape(NC * NS, EDGES_PER_TILE // CH, CH)
    degp = _deg_kernel(dstd, zdeg, ones_c)
    deg0 = degp[0, :, 0:1]
    deg1 = degp[1, :, 0:1]

    m, z = _tc_front(x_p, deg0, deg1, W1a, W1b, b1)
    for Wa, Wb, b in ((W2a, W2b, b2), (W3a, W3b, b3)):
        agg = _edge_kernel(m.reshape(NC * NP, HH), src4, dst3, zrows)
        m, z = _tc_mid(agg, z, deg0, deg1, Wa, Wb, b)
    agg = _edge_kernel(m.reshape(NC * NP, HH), src4, dst3, zrows)
    return _tc_final(agg, z, deg0, deg1, gid, Wfc, bfc, Wout, bout)


# speedup vs baseline: 1.1069x; 1.1069x over previous
import functools

import jax
import jax.numpy as jnp
from jax import lax
from jax.experimental import pallas as pl
from jax.experimental.pallas import tpu as pltpu
from jax.experimental.pallas import tpu_sc as plsc

N = 10000
E = 160000
F = 256
H = 256
HH = 128
P = 64
C = 2
G = 16

NC = 2
NS = 16
NP = 10240
EP = 163840
CH = 128
ROWS_PER_SUB = NP // NS
EDGES_PER_SUB = EP // NS
EDGES_PER_TILE = EP // (NC * NS)
NCH = EDGES_PER_SUB // CH
DW = 16

@functools.cache
def _sc_kernels():
    mesh = plsc.VectorSubcoreMesh(core_axis_name="c", subcore_axis_name="s",
                                  num_cores=NC, num_subcores=NS)

    @functools.partial(
        pl.kernel,
        out_type=jax.ShapeDtypeStruct((NC, NP, DW), jnp.float32),
        mesh=mesh,
        scratch_types=[
            pltpu.VMEM((EDGES_PER_TILE // CH, CH), jnp.int32),
            pltpu.VMEM((CH, DW), jnp.float32),
            pltpu.VMEM_SHARED((NP, DW), jnp.float32),
            pltpu.SemaphoreType.DMA,
            pltpu.SemaphoreType.DMA,
        ],
    )
    def _deg_kernel(dstd_hbm, zeros_hbm, ones_hbm, deg_out, didx_v, ones_v,
                    deg_sh, dsem0, dsem1):
        c = lax.axis_index("c")
        s = lax.axis_index("s")
        wid = s * NC + c
        pltpu.sync_copy(zeros_hbm.at[pl.ds(s * ROWS_PER_SUB, ROWS_PER_SUB)],
                        deg_sh.at[pl.ds(s * ROWS_PER_SUB, ROWS_PER_SUB)])
        pltpu.sync_copy(ones_hbm, ones_v)
        plsc.subcore_barrier()

        nd = EDGES_PER_TILE // CH
        pltpu.sync_copy(dstd_hbm.at[wid], didx_v)
        sd = [None] * nd
        for j in range(nd):
            if j >= 2:
                sd[j - 2].wait()
            sd[j] = pltpu.async_copy(ones_v, deg_sh.at[didx_v.at[j]],
                                     dsem0 if j % 2 == 0 else dsem1, add=True)
        sd[nd - 2].wait()
        sd[nd - 1].wait()
        plsc.subcore_barrier()
        pltpu.sync_copy(deg_sh.at[pl.ds(s * ROWS_PER_SUB, ROWS_PER_SUB)],
                        deg_out.at[c, pl.ds(s * ROWS_PER_SUB, ROWS_PER_SUB)])

    @functools.partial(
        pl.kernel,
        out_type=jax.ShapeDtypeStruct((NC, NP, HH), jnp.float32),
        mesh=mesh,
        scratch_types=[
            pltpu.VMEM((NCH // 2, CH), jnp.int32),
            pltpu.VMEM((NCH // 2, CH), jnp.int32),
            pltpu.VMEM((CH, HH), jnp.float32),
            pltpu.VMEM((CH, HH), jnp.float32),
            pltpu.VMEM_SHARED((NP, HH), jnp.float32),
            pltpu.SemaphoreType.DMA,
            pltpu.SemaphoreType.DMA,
            pltpu.SemaphoreType.DMA,
            pltpu.SemaphoreType.DMA,
        ],
    )
    def _edge_kernel(gtab_hbm, src4_hbm, dst3_hbm, zrows_hbm, agg_out,
                     sidx, didx, buf0, buf1, agg_sh, gsem0, gsem1, ssem0, ssem1):
        c = lax.axis_index("c")
        s = lax.axis_index("s")
        nh = NCH // 2
        bufs = (buf0, buf1)
        gsems = (gsem0, gsem1)
        ssems = (ssem0, ssem1)
        pltpu.sync_copy(zrows_hbm.at[pl.ds(s * ROWS_PER_SUB, ROWS_PER_SUB)],
                        agg_sh.at[pl.ds(s * ROWS_PER_SUB, ROWS_PER_SUB)])
        plsc.subcore_barrier()

        for half in range(2):
            pltpu.sync_copy(src4_hbm.at[c, s, pl.ds(half * nh, nh)], sidx)
            pltpu.sync_copy(dst3_hbm.at[s, pl.ds(half * nh, nh)], didx)
            gd = [None] * nh
            sd = [None] * nh
            for j in range(nh):
                b = j % 2
                if j >= 2:
                    sd[j - 2].wait()
                gd[j] = pltpu.async_copy(gtab_hbm.at[sidx.at[j]],
                                         bufs[b], gsems[b])
                if j >= 1:
                    gd[j - 1].wait()
                    sd[j - 1] = pltpu.async_copy(
                        bufs[1 - b], agg_sh.at[didx.at[j - 1]],
                        ssems[1 - b], add=True)
            gd[nh - 1].wait()
            sd[nh - 1] = pltpu.async_copy(
                bufs[(nh - 1) % 2], agg_sh.at[didx.at[nh - 1]],
                ssems[(nh - 1) % 2], add=True)
            sd[nh - 2].wait()
            sd[nh - 1].wait()
        plsc.subcore_barrier()
        pltpu.sync_copy(agg_sh.at[pl.ds(s * ROWS_PER_SUB, ROWS_PER_SUB)],
                        agg_out.at[c, pl.ds(s * ROWS_PER_SUB, ROWS_PER_SUB)])

    return _deg_kernel, _edge_kernel


BN = 512
NB = NP // BN


def _di_block(deg0_ref, deg1_ref):
    deg = deg0_ref[...] + deg1_ref[...]
    return lax.rsqrt(jnp.maximum(deg, 1.0))


def _front_body(x_ref, deg0_ref, deg1_ref, wa_ref, wb_ref, b_ref, m_ref, z_ref):
    di = _di_block(deg0_ref, deg1_ref)
    xb = x_ref[...]
    g = xb * di
    m_ref[0] = jnp.dot(g, wa_ref[:, :HH], preferred_element_type=jnp.float32)
    m_ref[1] = jnp.dot(g, wa_ref[:, HH:], preferred_element_type=jnp.float32)
    z_ref[...] = jnp.dot(xb, wb_ref[...], preferred_element_type=jnp.float32) + b_ref[...]


def _mid_body(agg_ref, z_ref, deg0_ref, deg1_ref, wa_ref, wb_ref, b_ref,
              m_ref, zn_ref):
    di = _di_block(deg0_ref, deg1_ref)
    a = jnp.concatenate([agg_ref[0], agg_ref[1]], axis=-1)
    h = jax.nn.relu(a * di + z_ref[...])
    g = h * di
    m_ref[0] = jnp.dot(g, wa_ref[:, :HH], preferred_element_type=jnp.float32)
    m_ref[1] = jnp.dot(g, wa_ref[:, HH:], preferred_element_type=jnp.float32)
    zn_ref[...] = jnp.dot(h, wb_ref[...], preferred_element_type=jnp.float32) + b_ref[...]


def _final_body(agg_ref, z_ref, deg0_ref, deg1_ref, gid_ref,
                wfc_ref, bfc_ref, wout_ref, bout_ref, out_ref, acc):
    n = pl.program_id(0)
    di = _di_block(deg0_ref, deg1_ref)
    a = jnp.concatenate([agg_ref[0], agg_ref[1]], axis=-1)
    h = jax.nn.relu(a * di + z_ref[...])
    gid = gid_ref[...]
    iota = lax.broadcasted_iota(jnp.int32, (G, BN), 0)
    oneh = (iota == gid).astype(jnp.float32)
    part = jnp.dot(oneh, h, preferred_element_type=jnp.float32)

    @pl.when(n == 0)
    def _():
        acc[...] = part

    @pl.when(n > 0)
    def _():
        acc[...] = acc[...] + part

    @pl.when(n == NB - 1)
    def _():
        pooled = acc[...]
        t = jnp.dot(pooled, wfc_ref[...], preferred_element_type=jnp.float32) + bfc_ref[...]
        t = jnp.dot(t, wout_ref[...], preferred_element_type=jnp.float32) + bout_ref[...]
        out_ref[...] = 1.0 / (1.0 + jnp.exp(-t))


def _node_spec(width):
    return pl.BlockSpec((BN, width), lambda n: (n, 0))


def _half_spec():
    return pl.BlockSpec((NC, BN, HH), lambda n: (0, n, 0))


def _full_spec(shape):
    nd = len(shape)
    return pl.BlockSpec(shape, lambda n: (0,) * nd)


def _tc_front(x_p, deg0, deg1, Wa, Wb, b):
    return pl.pallas_call(
        _front_body,
        grid=(NB,),
        in_specs=[
            _node_spec(F), _node_spec(1), _node_spec(1),
            _full_spec((F, H)), _full_spec((F, H)), _full_spec((1, H)),
        ],
        out_specs=[_half_spec(), _node_spec(H)],
        out_shape=[
            jax.ShapeDtypeStruct((NC, NP, HH), jnp.float32),
            jax.ShapeDtypeStruct((NP, H), jnp.float32),
        ],
    )(x_p, deg0, deg1, Wa, Wb, b.reshape(1, H))


def _tc_mid(agg, z, deg0, deg1, Wa, Wb, b):
    return pl.pallas_call(
        _mid_body,
        grid=(NB,),
        in_specs=[
            _half_spec(), _node_spec(H), _node_spec(1), _node_spec(1),
            _full_spec((H, H)), _full_spec((H, H)), _full_spec((1, H)),
        ],
        out_specs=[_half_spec(), _node_spec(H)],
        out_shape=[
            jax.ShapeDtypeStruct((NC, NP, HH), jnp.float32),
            jax.ShapeDtypeStruct((NP, H), jnp.float32),
        ],
    )(agg, z, deg0, deg1, Wa, Wb, b.reshape(1, H))


def _tc_final(agg, z, deg0, deg1, gid, Wfc, bfc, Wout, bout):
    return pl.pallas_call(
        _final_body,
        grid=(NB,),
        in_specs=[
            _half_spec(), _node_spec(H), _node_spec(1), _node_spec(1),
            pl.BlockSpec((1, BN), lambda n: (0, n)),
            _full_spec((H, P)), _full_spec((1, P)),
            _full_spec((P, C)), _full_spec((1, C)),
        ],
        out_specs=_full_spec((G, C)),
        out_shape=jax.ShapeDtypeStruct((G, C), jnp.float32),
        scratch_shapes=[pltpu.VMEM((G, H), jnp.float32)],
        compiler_params=pltpu.CompilerParams(
            dimension_semantics=("arbitrary",)),
    )(agg, z, deg0, deg1, gid, Wfc, bfc.reshape(1, P), Wout, bout.reshape(1, C))


def kernel(x, edge_index, i, W1a, W1b, b1, W2a, W2b, b2, W3a, W3b, b3,
           Wfc, bfc, Wout, bout):
    f32 = jnp.float32
    src = edge_index[0]
    dst = edge_index[1]
    pad_e = EP - E
    src_p = jnp.concatenate([src, jnp.zeros((pad_e,), jnp.int32)])
    dst_p = jnp.concatenate([dst, jnp.full((pad_e,), N, jnp.int32)])
    src4 = jnp.stack([src_p, src_p + NP]).reshape(NC, NS, NCH, CH)
    dst3 = dst_p.reshape(NS, NCH, CH)
    x_p = jnp.pad(x, ((0, NP - N), (0, 0)))
    gid = jnp.pad(i, (0, NP - N), constant_values=G).reshape(1, NP)

    zrows = jnp.zeros((NP, HH), f32)
    zdeg = jnp.zeros((NP, DW), f32)
    ones_c = jnp.ones((CH, DW), f32)

    _deg_kernel, _edge_kernel = _sc_kernels()
    dstd = dst_p.reshape(NC * NS, EDGES_PER_TILE // CH, CH)
    degp = _deg_kernel(dstd, zdeg, ones_c)
    deg0 = degp[0, :, 0:1]
    deg1 = degp[1, :, 0:1]

    m, z = _tc_front(x_p, deg0, deg1, W1a, W1b, b1)
    for Wa, Wb, b in ((W2a, W2b, b2), (W3a, W3b, b3)):
        agg = _edge_kernel(m.reshape(NC * NP, HH), src4, dst3, zrows)
        m, z = _tc_mid(agg, z, deg0, deg1, Wa, Wb, b)
    agg = _edge_kernel(m.reshape(NC * NP, HH), src4, dst3, zrows)
    return _tc_final(agg, z, deg0, deg1, gid, Wfc, bfc, Wout, bout)


# speedup vs baseline: 1.1117x; 1.0043x over previous
import functools

import jax
import jax.numpy as jnp
from jax import lax
from jax.experimental import pallas as pl
from jax.experimental.pallas import tpu as pltpu
from jax.experimental.pallas import tpu_sc as plsc

N = 10000
E = 160000
F = 256
H = 256
HH = 128
P = 64
C = 2
G = 16

NC = 2
NS = 16
NP = 10240
EP = 163840
CH = 128
ROWS_PER_SUB = NP // NS
EDGES_PER_SUB = EP // NS
EDGES_PER_TILE = EP // (NC * NS)
NCH = EDGES_PER_SUB // CH
DW = 16

@functools.cache
def _sc_kernels():
    mesh = plsc.VectorSubcoreMesh(core_axis_name="c", subcore_axis_name="s",
                                  num_cores=NC, num_subcores=NS)

    @functools.partial(
        pl.kernel,
        out_type=jax.ShapeDtypeStruct((NC, NP, DW), jnp.float32),
        mesh=mesh,
        scratch_types=[
            pltpu.VMEM((EDGES_PER_TILE // CH, CH), jnp.int32),
            pltpu.VMEM((CH, DW), jnp.float32),
            pltpu.VMEM_SHARED((NP, DW), jnp.float32),
            pltpu.SemaphoreType.DMA,
            pltpu.SemaphoreType.DMA,
        ],
    )
    def _deg_kernel(dstd_hbm, zeros_hbm, ones_hbm, deg_out, didx_v, ones_v,
                    deg_sh, dsem0, dsem1):
        c = lax.axis_index("c")
        s = lax.axis_index("s")
        wid = s * NC + c
        pltpu.sync_copy(zeros_hbm.at[pl.ds(s * ROWS_PER_SUB, ROWS_PER_SUB)],
                        deg_sh.at[pl.ds(s * ROWS_PER_SUB, ROWS_PER_SUB)])
        pltpu.sync_copy(ones_hbm, ones_v)
        plsc.subcore_barrier()

        nd = EDGES_PER_TILE // CH
        pltpu.sync_copy(dstd_hbm.at[wid], didx_v)
        sd = [None] * nd
        for j in range(nd):
            if j >= 2:
                sd[j - 2].wait()
            sd[j] = pltpu.async_copy(ones_v, deg_sh.at[didx_v.at[j]],
                                     dsem0 if j % 2 == 0 else dsem1, add=True)
        sd[nd - 2].wait()
        sd[nd - 1].wait()
        plsc.subcore_barrier()
        pltpu.sync_copy(deg_sh.at[pl.ds(s * ROWS_PER_SUB, ROWS_PER_SUB)],
                        deg_out.at[c, pl.ds(s * ROWS_PER_SUB, ROWS_PER_SUB)])

    @functools.partial(
        pl.kernel,
        out_type=jax.ShapeDtypeStruct((NC, NP, HH), jnp.float32),
        mesh=mesh,
        scratch_types=[
            pltpu.VMEM((NCH // 2, CH), jnp.int32),
            pltpu.VMEM((NCH // 2, CH), jnp.int32),
            pltpu.VMEM((CH, HH), jnp.float32),
            pltpu.VMEM((CH, HH), jnp.float32),
            pltpu.VMEM_SHARED((NP, HH), jnp.float32),
            pltpu.SemaphoreType.DMA,
            pltpu.SemaphoreType.DMA,
            pltpu.SemaphoreType.DMA,
            pltpu.SemaphoreType.DMA,
        ],
    )
    def _edge_kernel(gtab_hbm, src4_hbm, dst3_hbm, zrows_hbm, agg_out,
                     sidx, didx, buf0, buf1, agg_sh, gsem0, gsem1, ssem0, ssem1):
        c = lax.axis_index("c")
        s = lax.axis_index("s")
        nh = NCH // 2
        bufs = (buf0, buf1)
        gsems = (gsem0, gsem1)
        ssems = (ssem0, ssem1)
        pltpu.sync_copy(zrows_hbm.at[pl.ds(s * ROWS_PER_SUB, ROWS_PER_SUB)],
                        agg_sh.at[pl.ds(s * ROWS_PER_SUB, ROWS_PER_SUB)])
        pltpu.sync_copy(src4_hbm.at[c, s, pl.ds(0, nh)], sidx)
        pltpu.sync_copy(dst3_hbm.at[s, pl.ds(0, nh)], didx)
        pre_gd = [pltpu.async_copy(gtab_hbm.at[sidx.at[0]], buf0, gsem0),
                  pltpu.async_copy(gtab_hbm.at[sidx.at[1]], buf1, gsem1)]
        plsc.subcore_barrier()

        for half in range(2):
            if half == 1:
                pltpu.sync_copy(src4_hbm.at[c, s, pl.ds(half * nh, nh)], sidx)
                pltpu.sync_copy(dst3_hbm.at[s, pl.ds(half * nh, nh)], didx)
            gd = [None] * nh
            sd = [None] * nh
            for j in range(nh):
                b = j % 2
                if j >= 2:
                    sd[j - 2].wait()
                if half == 0 and j < 2:
                    gd[j] = pre_gd[j]
                else:
                    gd[j] = pltpu.async_copy(gtab_hbm.at[sidx.at[j]],
                                             bufs[b], gsems[b])
                if j >= 1:
                    gd[j - 1].wait()
                    sd[j - 1] = pltpu.async_copy(
                        bufs[1 - b], agg_sh.at[didx.at[j - 1]],
                        ssems[1 - b], add=True)
            gd[nh - 1].wait()
            sd[nh - 1] = pltpu.async_copy(
                bufs[(nh - 1) % 2], agg_sh.at[didx.at[nh - 1]],
                ssems[(nh - 1) % 2], add=True)
            sd[nh - 2].wait()
            sd[nh - 1].wait()
        plsc.subcore_barrier()
        pltpu.sync_copy(agg_sh.at[pl.ds(s * ROWS_PER_SUB, ROWS_PER_SUB)],
                        agg_out.at[c, pl.ds(s * ROWS_PER_SUB, ROWS_PER_SUB)])

    return _deg_kernel, _edge_kernel


BN = 512
NB = NP // BN


def _di_block(deg0_ref, deg1_ref):
    deg = deg0_ref[...] + deg1_ref[...]
    return lax.rsqrt(jnp.maximum(deg, 1.0))


def _front_body(x_ref, deg0_ref, deg1_ref, wa_ref, wb_ref, b_ref, m_ref, z_ref):
    di = _di_block(deg0_ref, deg1_ref)
    xb = x_ref[...]
    g = xb * di
    m_ref[0] = jnp.dot(g, wa_ref[:, :HH], preferred_element_type=jnp.float32)
    m_ref[1] = jnp.dot(g, wa_ref[:, HH:], preferred_element_type=jnp.float32)
    z_ref[...] = jnp.dot(xb, wb_ref[...], preferred_element_type=jnp.float32) + b_ref[...]


def _mid_body(agg_ref, z_ref, deg0_ref, deg1_ref, wa_ref, wb_ref, b_ref,
              m_ref, zn_ref):
    di = _di_block(deg0_ref, deg1_ref)
    a = jnp.concatenate([agg_ref[0], agg_ref[1]], axis=-1)
    h = jax.nn.relu(a * di + z_ref[...])
    g = h * di
    m_ref[0] = jnp.dot(g, wa_ref[:, :HH], preferred_element_type=jnp.float32)
    m_ref[1] = jnp.dot(g, wa_ref[:, HH:], preferred_element_type=jnp.float32)
    zn_ref[...] = jnp.dot(h, wb_ref[...], preferred_element_type=jnp.float32) + b_ref[...]


def _final_body(agg_ref, z_ref, deg0_ref, deg1_ref, gid_ref,
                wfc_ref, bfc_ref, wout_ref, bout_ref, out_ref, acc):
    n = pl.program_id(0)
    di = _di_block(deg0_ref, deg1_ref)
    a = jnp.concatenate([agg_ref[0], agg_ref[1]], axis=-1)
    h = jax.nn.relu(a * di + z_ref[...])
    gid = gid_ref[...]
    iota = lax.broadcasted_iota(jnp.int32, (G, BN), 0)
    oneh = (iota == gid).astype(jnp.float32)
    part = jnp.dot(oneh, h, preferred_element_type=jnp.float32)

    @pl.when(n == 0)
    def _():
        acc[...] = part

    @pl.when(n > 0)
    def _():
        acc[...] = acc[...] + part

    @pl.when(n == NB - 1)
    def _():
        pooled = acc[...]
        t = jnp.dot(pooled, wfc_ref[...], preferred_element_type=jnp.float32) + bfc_ref[...]
        t = jnp.dot(t, wout_ref[...], preferred_element_type=jnp.float32) + bout_ref[...]
        out_ref[...] = 1.0 / (1.0 + jnp.exp(-t))


def _node_spec(width):
    return pl.BlockSpec((BN, width), lambda n: (n, 0))


def _half_spec():
    return pl.BlockSpec((NC, BN, HH), lambda n: (0, n, 0))


def _full_spec(shape):
    nd = len(shape)
    return pl.BlockSpec(shape, lambda n: (0,) * nd)


def _tc_front(x_p, deg0, deg1, Wa, Wb, b):
    return pl.pallas_call(
        _front_body,
        grid=(NB,),
        in_specs=[
            _node_spec(F), _node_spec(1), _node_spec(1),
            _full_spec((F, H)), _full_spec((F, H)), _full_spec((1, H)),
        ],
        out_specs=[_half_spec(), _node_spec(H)],
        out_shape=[
            jax.ShapeDtypeStruct((NC, NP, HH), jnp.float32),
            jax.ShapeDtypeStruct((NP, H), jnp.float32),
        ],
    )(x_p, deg0, deg1, Wa, Wb, b.reshape(1, H))


def _tc_mid(agg, z, deg0, deg1, Wa, Wb, b):
    return pl.pallas_call(
        _mid_body,
        grid=(NB,),
        in_specs=[
            _half_spec(), _node_spec(H), _node_spec(1), _node_spec(1),
            _full_spec((H, H)), _full_spec((H, H)), _full_spec((1, H)),
        ],
        out_specs=[_half_spec(), _node_spec(H)],
        out_shape=[
            jax.ShapeDtypeStruct((NC, NP, HH), jnp.float32),
            jax.ShapeDtypeStruct((NP, H), jnp.float32),
        ],
    )(agg, z, deg0, deg1, Wa, Wb, b.reshape(1, H))


def _tc_final(agg, z, deg0, deg1, gid, Wfc, bfc, Wout, bout):
    return pl.pallas_call(
        _final_body,
        grid=(NB,),
        in_specs=[
            _half_spec(), _node_spec(H), _node_spec(1), _node_spec(1),
            pl.BlockSpec((1, BN), lambda n: (0, n)),
            _full_spec((H, P)), _full_spec((1, P)),
            _full_spec((P, C)), _full_spec((1, C)),
        ],
        out_specs=_full_spec((G, C)),
        out_shape=jax.ShapeDtypeStruct((G, C), jnp.float32),
        scratch_shapes=[pltpu.VMEM((G, H), jnp.float32)],
        compiler_params=pltpu.CompilerParams(
            dimension_semantics=("arbitrary",)),
    )(agg, z, deg0, deg1, gid, Wfc, bfc.reshape(1, P), Wout, bout.reshape(1, C))


def kernel(x, edge_index, i, W1a, W1b, b1, W2a, W2b, b2, W3a, W3b, b3,
           Wfc, bfc, Wout, bout):
    f32 = jnp.float32
    src = edge_index[0]
    dst = edge_index[1]
    pad_e = EP - E
    src_p = jnp.concatenate([src, jnp.zeros((pad_e,), jnp.int32)])
    dst_p = jnp.concatenate([dst, jnp.full((pad_e,), N, jnp.int32)])
    src4 = jnp.stack([src_p, src_p + NP]).reshape(NC, NS, NCH, CH)
    dst3 = dst_p.reshape(NS, NCH, CH)
    x_p = jnp.pad(x, ((0, NP - N), (0, 0)))
    gid = jnp.pad(i, (0, NP - N), constant_values=G).reshape(1, NP)

    zrows = jnp.zeros((NP, HH), f32)
    zdeg = jnp.zeros((NP, DW), f32)
    ones_c = jnp.ones((CH, DW), f32)

    _deg_kernel, _edge_kernel = _sc_kernels()
    dstd = dst_p.reshape(NC * NS, EDGES_PER_TILE // CH, CH)
    degp = _deg_kernel(dstd, zdeg, ones_c)
    deg0 = degp[0, :, 0:1]
    deg1 = degp[1, :, 0:1]

    m, z = _tc_front(x_p, deg0, deg1, W1a, W1b, b1)
    for Wa, Wb, b in ((W2a, W2b, b2), (W3a, W3b, b3)):
        agg = _edge_kernel(m.reshape(NC * NP, HH), src4, dst3, zrows)
        m, z = _tc_mid(agg, z, deg0, deg1, Wa, Wb, b)
    agg = _edge_kernel(m.reshape(NC * NP, HH), src4, dst3, zrows)
    return _tc_final(agg, z, deg0, deg1, gid, Wfc, bfc, Wout, bout)
